# Initial kernel scaffold; baseline (speedup 1.0000x reference)
import jax
import jax.numpy as jnp
from jax.experimental import pallas as pl


def kernel(proposals, batch_indices, gt_boxes, gt_labels, gt_masks):
    raise NotImplementedError("write your pallas kernel here")


# speedup vs baseline: 920.1815x; 920.1815x over previous
import jax
import jax.numpy as jnp
from jax.experimental import pallas as pl

TRAIN_ROIS = 256
MASK_POOL = 28
POS_THR = 0.1
NEG_THR = 0.02
POS_NUM = 64
NEG_NUM = TRAIN_ROIS - POS_NUM


def _score_kernel(props_ref, bidx_ref, gt_ref, pos_ref, neg_ref):
    i = pl.program_id(0)
    props = props_ref[...]
    bidx = bidx_ref[...]
    gt = gt_ref[0]
    mask = bidx == i

    r0 = props[0:1]
    r1 = props[1:2]
    r2 = props[2:3]
    r3 = props[3:4]
    r4 = props[4:5]
    r5 = props[5:6]
    b0 = gt[:, 0:1]
    b1 = gt[:, 1:2]
    b2 = gt[:, 2:3]
    b3 = gt[:, 3:4]
    b4 = gt[:, 4:5]
    b5 = gt[:, 5:6]

    iy = jnp.maximum(jnp.minimum(b3, r3) - jnp.maximum(b0, r0), 0.0)
    ix = jnp.maximum(jnp.minimum(b4, r4) - jnp.maximum(b1, r1), 0.0)
    iz = jnp.maximum(jnp.minimum(b5, r5) - jnp.maximum(b2, r2), 0.0)
    inter = iy * ix * iz
    vb = (b3 - b0) * (b4 - b1) * (b5 - b2)
    vr = (r3 - r0) * (r4 - r1) * (r5 - r2)
    union = vb + vr - inter
    iou = inter / (union + 1e-8)
    iou = jnp.where(mask, iou, -1.0)
    roi_max = jnp.max(iou, axis=0, keepdims=True)
    best = (iou == roi_max) & mask
    pos_ref[0] = jnp.where(best & (iou >= POS_THR), iou, -1e9)
    neg_ref[0] = jnp.where(best & (iou < NEG_THR) & (iou >= 0.0), -iou, -1e9)


def _crop_kernel(rois_ref, boxes_ref, vol_ref, masks_ref, deltas_ref):
    roi = rois_ref[0]
    box = boxes_ref[0]
    vol = vol_ref[0]
    V = vol.shape[0]
    Pn = MASK_POOL
    eps = 1e-8

    sz = roi[:, 3:6] - roi[:, 0:3]
    ctr = roi[:, 0:3] + 0.5 * sz
    gsz = box[:, 3:6] - box[:, 0:3]
    gctr = box[:, 0:3] + 0.5 * gsz
    deltas_ref[0] = jnp.concatenate(
        [(gctr - ctr) / (sz + eps), jnp.log((gsz + eps) / (sz + eps))], axis=1)

    idx = jax.lax.broadcasted_iota(jnp.int32, (Pn, 1), 0).astype(jnp.float32)
    cols = jax.lax.broadcasted_iota(jnp.int32, (1, V), 1).astype(jnp.float32)

    def weights(lo, hi):
        c = (lo + idx * ((hi - lo) * (1.0 / (Pn - 1)))) * float(V - 1)
        c = jnp.clip(c, 0.0, float(V - 1))
        return jnp.maximum(1.0 - jnp.abs(c - cols), 0.0)

    wy = weights(roi[0:1, 0:1], roi[0:1, 3:4])
    wx = weights(roi[0:1, 1:2], roi[0:1, 4:5])
    wz = weights(roi[0:1, 2:3], roi[0:1, 5:6])

    t1 = jnp.dot(wy, vol.reshape(V, V * V),
                 preferred_element_type=jnp.float32)
    t1 = t1.reshape(Pn, V, V).transpose(0, 2, 1).reshape(Pn * V, V)
    t2 = jnp.dot(t1, wx.T, preferred_element_type=jnp.float32)
    t2 = t2.reshape(Pn, V, Pn).transpose(0, 2, 1).reshape(Pn * Pn, V)
    t3 = jnp.dot(t2, wz.T, preferred_element_type=jnp.float32)
    masks_ref[0] = t3.reshape(Pn, Pn, Pn)


@jax.jit
def kernel(proposals, batch_indices, gt_boxes, gt_labels, gt_masks):
    B, NGT = gt_boxes.shape[:2]
    P = proposals.shape[0]
    V = gt_masks.shape[1]
    R = B * TRAIN_ROIS

    props_t = proposals.T
    bidx = batch_indices.astype(jnp.int32).reshape(1, P)

    pos, neg = pl.pallas_call(
        _score_kernel,
        grid=(B,),
        in_specs=[
            pl.BlockSpec((6, P), lambda i: (0, 0)),
            pl.BlockSpec((1, P), lambda i: (0, 0)),
            pl.BlockSpec((1, NGT, 6), lambda i: (i, 0, 0)),
        ],
        out_specs=[
            pl.BlockSpec((1, NGT, P), lambda i: (i, 0, 0)),
            pl.BlockSpec((1, NGT, P), lambda i: (i, 0, 0)),
        ],
        out_shape=[
            jax.ShapeDtypeStruct((B, NGT, P), jnp.float32),
            jax.ShapeDtypeStruct((B, NGT, P), jnp.float32),
        ],
    )(props_t, bidx, gt_boxes)

    _, pos_flat = jax.lax.top_k(pos.reshape(B, NGT * P), POS_NUM)
    _, neg_flat = jax.lax.top_k(neg.reshape(B, NGT * P), NEG_NUM)
    flat = jnp.concatenate([pos_flat, neg_flat], axis=1)
    gt_idx = flat // P
    roi_idx = flat % P

    rois = jnp.take(proposals, roi_idx.reshape(-1), axis=0)
    boxes = jnp.take_along_axis(gt_boxes, gt_idx[:, :, None], axis=1)
    labels = jnp.take_along_axis(gt_labels, gt_idx, axis=1)
    labels = labels.at[:, POS_NUM:].set(0).reshape(-1)
    rois_indices = jnp.repeat(
        jnp.arange(B, dtype=jnp.int32), TRAIN_ROIS, total_repeat_length=R)

    masks, deltas = pl.pallas_call(
        _crop_kernel,
        grid=(R,),
        in_specs=[
            pl.BlockSpec((1, 1, 6), lambda r: (r, 0, 0)),
            pl.BlockSpec((1, 1, 6), lambda r: (r, 0, 0)),
            pl.BlockSpec((1, V, V, V), lambda r: (r // TRAIN_ROIS, 0, 0, 0)),
        ],
        out_specs=[
            pl.BlockSpec((1, MASK_POOL, MASK_POOL, MASK_POOL),
                         lambda r: (r, 0, 0, 0)),
            pl.BlockSpec((1, 1, 6), lambda r: (r, 0, 0)),
        ],
        out_shape=[
            jax.ShapeDtypeStruct((R, MASK_POOL, MASK_POOL, MASK_POOL),
                                 jnp.float32),
            jax.ShapeDtypeStruct((R, 1, 6), jnp.float32),
        ],
    )(rois.reshape(R, 1, 6), boxes.reshape(R, 1, 6), gt_masks)

    return (rois, deltas.reshape(R, 6), labels, masks, rois_indices)
